# baseline (speedup 1.0000x reference)
import functools

import jax
import jax.numpy as jnp
from jax import lax
from jax.experimental import pallas as pl
from jax.experimental.pallas import tpu as pltpu
from jax.experimental.pallas import tpu_sc as plsc

T, H, I, E, K = 2048, 2048, 1536, 8, 2
TILE = 256
NT = (T * K) // TILE + E
GP = NT * TILE
NC, NS = 2, 16
NW = NC * NS


def _k1_body(x_ref, gw_ref, sg_ref, su_ref, sd_ref, sh_ref, cmb_ref, sel_ref):
    xf = x_ref[...]
    xb = xf.astype(jnp.bfloat16)
    g = lax.dot_general(xb, sg_ref[...], (((1,), (1,)), ((), ())),
                        preferred_element_type=jnp.float32)
    u = lax.dot_general(xb, su_ref[...], (((1,), (1,)), ((), ())),
                        preferred_element_type=jnp.float32)
    h = (g * jax.nn.sigmoid(g) * u).astype(jnp.bfloat16)
    sh_ref[...] = lax.dot_general(h, sd_ref[...], (((1,), (1,)), ((), ())),
                                  preferred_element_type=jnp.float32)

    logits = lax.dot_general(xb, gw_ref[...].astype(jnp.bfloat16),
                             (((1,), (1,)), ((), ())),
                             preferred_element_type=jnp.float32)
    idx = lax.broadcasted_iota(jnp.int32, (TILE, E), 1)
    m1 = jnp.max(logits, axis=1, keepdims=True)
    i1 = jnp.min(jnp.where(logits == m1, idx, E + 1), axis=1, keepdims=True)
    l2 = jnp.where(idx == i1, -jnp.inf, logits)
    m2 = jnp.max(l2, axis=1, keepdims=True)
    i2 = jnp.min(jnp.where(l2 == m2, idx, E + 1), axis=1, keepdims=True)
    w1 = 1.0 / (1.0 + jnp.exp(m2 - m1))
    w2 = 1.0 - w1
    on1 = (idx == i1)
    on2 = (idx == i2)
    cmb_ref[...] = jnp.where(on1, w1, 0.0) + jnp.where(on2, w2, 0.0)
    sel_ref[...] = jnp.where(on1 | on2, 1.0, 0.0)


def _k1(x, gate_w, sg_b, su_b, sd_b):
    return pl.pallas_call(
        _k1_body,
        grid=(T // TILE,),
        in_specs=[
            pl.BlockSpec((TILE, H), lambda i: (i, 0)),
            pl.BlockSpec((E, H), lambda i: (0, 0)),
            pl.BlockSpec((I, H), lambda i: (0, 0)),
            pl.BlockSpec((I, H), lambda i: (0, 0)),
            pl.BlockSpec((H, I), lambda i: (0, 0)),
        ],
        out_specs=[
            pl.BlockSpec((TILE, H), lambda i: (i, 0)),
            pl.BlockSpec((TILE, E), lambda i: (i, 0)),
            pl.BlockSpec((TILE, E), lambda i: (i, 0)),
        ],
        out_shape=[
            jax.ShapeDtypeStruct((T, H), jnp.float32),
            jax.ShapeDtypeStruct((T, E), jnp.float32),
            jax.ShapeDtypeStruct((T, E), jnp.float32),
        ],
    )(x, gate_w, sg_b, su_b, sd_b)


def _k3_body(tid_ref, xs_ref, wg_ref, wu_ref, wd_ref, wr_ref, ys_ref):
    xb = xs_ref[...].astype(jnp.bfloat16)
    g = lax.dot_general(xb, wg_ref[0], (((1,), (1,)), ((), ())),
                        preferred_element_type=jnp.float32)
    u = lax.dot_general(xb, wu_ref[0], (((1,), (1,)), ((), ())),
                        preferred_element_type=jnp.float32)
    h = (g * jax.nn.sigmoid(g) * u).astype(jnp.bfloat16)
    y = lax.dot_general(h, wd_ref[0], (((1,), (1,)), ((), ())),
                        preferred_element_type=jnp.float32)
    ys_ref[...] = y * wr_ref[0, 0, :][:, None]


def _k3(tid, xs, wg_b, wu_b, wd_b, wrow3):
    spec = pltpu.PrefetchScalarGridSpec(
        num_scalar_prefetch=1,
        grid=(NT,),
        in_specs=[
            pl.BlockSpec((TILE, H), lambda i, tid: (i, 0)),
            pl.BlockSpec((1, I, H), lambda i, tid: (tid[i], 0, 0)),
            pl.BlockSpec((1, I, H), lambda i, tid: (tid[i], 0, 0)),
            pl.BlockSpec((1, H, I), lambda i, tid: (tid[i], 0, 0)),
            pl.BlockSpec((1, 1, TILE), lambda i, tid: (i, 0, 0)),
        ],
        out_specs=pl.BlockSpec((TILE, H), lambda i, tid: (i, 0)),
    )
    return pl.pallas_call(
        _k3_body,
        grid_spec=spec,
        out_shape=jax.ShapeDtypeStruct((GP, H), jnp.float32),
    )(tid, xs, wg_b, wu_b, wd_b, wrow3)


def _sc_gather(x, tok_ids):
    rpw = GP // NW
    ch = 32
    nchunk = rpw // ch
    mesh = plsc.VectorSubcoreMesh(core_axis_name="c", subcore_axis_name="s")

    @functools.partial(
        pl.kernel, mesh=mesh,
        out_type=jax.ShapeDtypeStruct((GP, H), jnp.float32),
        scratch_types=[
            pltpu.VMEM((ch,), jnp.int32),
            pltpu.VMEM((ch, H), jnp.float32),
            pltpu.SemaphoreType.DMA,
        ],
    )
    def k(x_hbm, ids_hbm, xs_hbm, idx_v, rows_v, sem):
        wid = lax.axis_index("s") * NC + lax.axis_index("c")
        base = wid * rpw

        def chunk(c, carry):
            b = pl.multiple_of(base + c * ch, ch)
            pltpu.sync_copy(ids_hbm.at[pl.ds(b, ch)], idx_v)
            pltpu.async_copy(x_hbm.at[idx_v], rows_v, sem).wait()
            pltpu.sync_copy(rows_v, xs_hbm.at[pl.ds(b, ch)])
            return carry

        lax.fori_loop(0, nchunk, chunk, 0)

    return k(x, tok_ids)


def _sc_combine(shared, ys, pos0, pos1):
    tpw = T // NW
    ch = 16
    nchunk = tpw // ch
    ncol = H // 16
    mesh = plsc.VectorSubcoreMesh(core_axis_name="c", subcore_axis_name="s")

    @functools.partial(
        pl.kernel, mesh=mesh,
        out_type=jax.ShapeDtypeStruct((T, H), jnp.float32),
        scratch_types=[
            pltpu.VMEM((ch,), jnp.int32),
            pltpu.VMEM((ch,), jnp.int32),
            pltpu.VMEM((ch, H), jnp.float32),
            pltpu.VMEM((ch, H), jnp.float32),
            pltpu.SemaphoreType.DMA,
        ],
    )
    def k(sh_hbm, ys_hbm, p0_hbm, p1_hbm, out_hbm, i0_v, i1_v, acc_v, buf_v, sem):
        wid = lax.axis_index("s") * NC + lax.axis_index("c")
        base = wid * tpw

        def add_buf(_):
            def row(r, carry):
                def col(j, carry2):
                    s = pl.ds(j * 16, 16)
                    acc_v[r, s] = acc_v[r, s] + buf_v[r, s]
                    return carry2
                return lax.fori_loop(0, ncol, col, carry)
            lax.fori_loop(0, ch, row, 0)

        def chunk(c, carry):
            b = pl.multiple_of(base + c * ch, ch)
            pltpu.sync_copy(sh_hbm.at[pl.ds(b, ch)], acc_v)
            pltpu.sync_copy(p0_hbm.at[pl.ds(b, ch)], i0_v)
            pltpu.sync_copy(p1_hbm.at[pl.ds(b, ch)], i1_v)
            pltpu.async_copy(ys_hbm.at[i0_v], buf_v, sem).wait()
            add_buf(None)
            pltpu.async_copy(ys_hbm.at[i1_v], buf_v, sem).wait()
            add_buf(None)
            pltpu.sync_copy(acc_v, out_hbm.at[pl.ds(b, ch)])
            return carry

        lax.fori_loop(0, nchunk, chunk, 0)

    return k(shared, ys, pos0, pos1)


def _route(combine, selm):
    sel = selm > 0.5
    selj = sel.astype(jnp.int32)
    cnt_excl = jnp.cumsum(selj, axis=0) - selj
    counts = jnp.sum(selj, axis=0)
    tiles_e = (counts + TILE - 1) // TILE
    tile_end = jnp.cumsum(tiles_e)
    seg_start = (tile_end - tiles_e) * TILE
    dest = seg_start[None, :] + cnt_excl
    dest_full = jnp.where(sel, dest, GP)

    tokens = jnp.broadcast_to(jnp.arange(T, dtype=jnp.int32)[:, None], (T, E))
    flat_dest = dest_full.ravel()
    tok_ids = jnp.zeros((GP + 1,), jnp.int32).at[flat_dest].set(tokens.ravel())[:GP]
    wrow = jnp.zeros((GP + 1,), jnp.float32).at[flat_dest].set(combine.ravel())[:GP]

    pos2 = jnp.sort(dest_full, axis=1)[:, :K]
    pos0 = pos2[:, 0].astype(jnp.int32)
    pos1 = pos2[:, 1].astype(jnp.int32)

    tnums = jnp.arange(NT, dtype=jnp.int32)
    tid_raw = jnp.searchsorted(tile_end, tnums, side="right").astype(jnp.int32)
    used = tile_end[E - 1]
    last_tid = tid_raw[used - 1]
    tid = jnp.where(tnums < used, tid_raw, last_tid)
    return tok_ids, wrow.reshape(NT, 1, TILE), pos0, pos1, tid


def kernel(x, gate_w, Wg, Wu, Wd, sg, su, sd):
    sg_b = sg.astype(jnp.bfloat16)
    su_b = su.astype(jnp.bfloat16)
    sd_b = sd.astype(jnp.bfloat16)
    wg_b = Wg.astype(jnp.bfloat16)
    wu_b = Wu.astype(jnp.bfloat16)
    wd_b = Wd.astype(jnp.bfloat16)

    shared, combine, selm = _k1(x, gate_w, sg_b, su_b, sd_b)
    tok_ids, wrow3, pos0, pos1, tid = _route(combine, selm)
    xs = _sc_gather(x, tok_ids)
    ys = _k3(tid, xs, wg_b, wu_b, wd_b, wrow3)
    return _sc_combine(shared, ys, pos0, pos1)


# speedup vs baseline: 1.0332x; 1.0332x over previous
import functools

import jax
import jax.numpy as jnp
from jax import lax
from jax.experimental import pallas as pl
from jax.experimental.pallas import tpu as pltpu
from jax.experimental.pallas import tpu_sc as plsc

T, H, I, E, K = 2048, 2048, 1536, 8, 2
TILE = 256
NT = (T * K) // TILE + E
GP = NT * TILE
NC, NS = 2, 16
NW = NC * NS


def _k1_body(x_ref, gw_ref, sg_ref, su_ref, sd_ref, sh_ref, cmb_ref, sel_ref):
    xf = x_ref[...]
    xb = xf.astype(jnp.bfloat16)
    g = lax.dot_general(xb, sg_ref[...], (((1,), (1,)), ((), ())),
                        preferred_element_type=jnp.float32)
    u = lax.dot_general(xb, su_ref[...], (((1,), (1,)), ((), ())),
                        preferred_element_type=jnp.float32)
    h = (g * jax.nn.sigmoid(g) * u).astype(jnp.bfloat16)
    sh_ref[...] = lax.dot_general(h, sd_ref[...], (((1,), (1,)), ((), ())),
                                  preferred_element_type=jnp.float32)

    logits = lax.dot_general(xb, gw_ref[...].astype(jnp.bfloat16),
                             (((1,), (1,)), ((), ())),
                             preferred_element_type=jnp.float32)
    idx = lax.broadcasted_iota(jnp.int32, (TILE, E), 1)
    m1 = jnp.max(logits, axis=1, keepdims=True)
    i1 = jnp.min(jnp.where(logits == m1, idx, E + 1), axis=1, keepdims=True)
    l2 = jnp.where(idx == i1, -jnp.inf, logits)
    m2 = jnp.max(l2, axis=1, keepdims=True)
    i2 = jnp.min(jnp.where(l2 == m2, idx, E + 1), axis=1, keepdims=True)
    w1 = 1.0 / (1.0 + jnp.exp(m2 - m1))
    w2 = 1.0 - w1
    on1 = (idx == i1)
    on2 = (idx == i2)
    cmb_ref[...] = jnp.where(on1, w1, 0.0) + jnp.where(on2, w2, 0.0)
    sel_ref[...] = jnp.where(on1 | on2, 1.0, 0.0)


def _k1(x, gate_w, sg_b, su_b, sd_b):
    return pl.pallas_call(
        _k1_body,
        grid=(T // TILE,),
        in_specs=[
            pl.BlockSpec((TILE, H), lambda i: (i, 0)),
            pl.BlockSpec((E, H), lambda i: (0, 0)),
            pl.BlockSpec((I, H), lambda i: (0, 0)),
            pl.BlockSpec((I, H), lambda i: (0, 0)),
            pl.BlockSpec((H, I), lambda i: (0, 0)),
        ],
        out_specs=[
            pl.BlockSpec((TILE, H), lambda i: (i, 0)),
            pl.BlockSpec((TILE, E), lambda i: (i, 0)),
            pl.BlockSpec((TILE, E), lambda i: (i, 0)),
        ],
        out_shape=[
            jax.ShapeDtypeStruct((T, H), jnp.float32),
            jax.ShapeDtypeStruct((T, E), jnp.float32),
            jax.ShapeDtypeStruct((T, E), jnp.float32),
        ],
    )(x, gate_w, sg_b, su_b, sd_b)


def _k3_body(tid_ref, xs_ref, wg_ref, wu_ref, wd_ref, wr_ref, ys_ref):
    xb = xs_ref[...].astype(jnp.bfloat16)
    g = lax.dot_general(xb, wg_ref[0], (((1,), (1,)), ((), ())),
                        preferred_element_type=jnp.float32)
    u = lax.dot_general(xb, wu_ref[0], (((1,), (1,)), ((), ())),
                        preferred_element_type=jnp.float32)
    h = (g * jax.nn.sigmoid(g) * u).astype(jnp.bfloat16)
    y = lax.dot_general(h, wd_ref[0], (((1,), (1,)), ((), ())),
                        preferred_element_type=jnp.float32)
    ys_ref[...] = y * wr_ref[0, 0, :][:, None]


def _k3(tid, xs, wg_b, wu_b, wd_b, wrow3):
    spec = pltpu.PrefetchScalarGridSpec(
        num_scalar_prefetch=1,
        grid=(NT,),
        in_specs=[
            pl.BlockSpec((TILE, H), lambda i, tid: (i, 0)),
            pl.BlockSpec((1, I, H), lambda i, tid: (tid[i], 0, 0)),
            pl.BlockSpec((1, I, H), lambda i, tid: (tid[i], 0, 0)),
            pl.BlockSpec((1, H, I), lambda i, tid: (tid[i], 0, 0)),
            pl.BlockSpec((1, 1, TILE), lambda i, tid: (i, 0, 0)),
        ],
        out_specs=pl.BlockSpec((TILE, H), lambda i, tid: (i, 0)),
    )
    return pl.pallas_call(
        _k3_body,
        grid_spec=spec,
        out_shape=jax.ShapeDtypeStruct((GP, H), jnp.float32),
    )(tid, xs, wg_b, wu_b, wd_b, wrow3)


def _sc_gather(x, tok_ids3):
    ch = 24
    nchunk = (GP // NW) // ch
    mesh = plsc.VectorSubcoreMesh(core_axis_name="c", subcore_axis_name="s")

    @functools.partial(
        pl.kernel, mesh=mesh,
        out_type=jax.ShapeDtypeStruct((GP, H), jnp.float32),
        scratch_types=[
            pltpu.VMEM((nchunk, ch), jnp.int32),
            pltpu.VMEM((ch, H), jnp.float32),
            pltpu.VMEM((ch, H), jnp.float32),
            pltpu.SemaphoreType.DMA,
            pltpu.SemaphoreType.DMA,
            pltpu.SemaphoreType.DMA,
            pltpu.SemaphoreType.DMA,
        ],
    )
    def k(x_hbm, ids_hbm, xs_hbm, idx_v, rows0, rows1, g0, g1, w0, w1):
        wid = lax.axis_index("s") * NC + lax.axis_index("c")
        base = wid * (nchunk * ch)
        bufs, gsems, wsems = (rows0, rows1), (g0, g1), (w0, w1)

        pltpu.sync_copy(ids_hbm.at[wid], idx_v)
        pltpu.async_copy(x_hbm.at[idx_v.at[0]], bufs[0], gsems[0])
        for c in range(nchunk):
            pltpu.make_async_copy(x_hbm.at[idx_v.at[c]], bufs[c % 2],
                                  gsems[c % 2]).wait()
            if c + 1 < nchunk:
                if c >= 1:
                    pltpu.make_async_copy(
                        bufs[(c + 1) % 2],
                        xs_hbm.at[pl.ds(base + (c - 1) * ch, ch)],
                        wsems[(c + 1) % 2]).wait()
                pltpu.async_copy(x_hbm.at[idx_v.at[c + 1]], bufs[(c + 1) % 2],
                                 gsems[(c + 1) % 2])
            pltpu.async_copy(bufs[c % 2], xs_hbm.at[pl.ds(base + c * ch, ch)],
                             wsems[c % 2])
        for c in (nchunk - 2, nchunk - 1):
            pltpu.make_async_copy(bufs[c % 2],
                                  xs_hbm.at[pl.ds(base + c * ch, ch)],
                                  wsems[c % 2]).wait()

    return k(x, tok_ids3)


def _sc_combine(shared, ys, pos0_3, pos1_3):
    ch = 8
    nchunk = (T // NW) // ch
    ncol = H // 16
    mesh = plsc.VectorSubcoreMesh(core_axis_name="c", subcore_axis_name="s")

    @functools.partial(
        pl.kernel, mesh=mesh,
        out_type=jax.ShapeDtypeStruct((T, H), jnp.float32),
        scratch_types=[
            pltpu.VMEM((nchunk, ch), jnp.int32),
            pltpu.VMEM((nchunk, ch), jnp.int32),
            pltpu.VMEM((ch, H), jnp.float32),
            pltpu.VMEM((ch, H), jnp.float32),
            pltpu.VMEM((ch, H), jnp.float32),
            pltpu.VMEM((ch, H), jnp.float32),
            pltpu.SemaphoreType.DMA,
            pltpu.SemaphoreType.DMA,
            pltpu.SemaphoreType.DMA,
            pltpu.SemaphoreType.DMA,
            pltpu.SemaphoreType.DMA,
            pltpu.SemaphoreType.DMA,
        ],
    )
    def k(sh_hbm, ys_hbm, p0_hbm, p1_hbm, out_hbm, i0_v, i1_v,
          acc0, acc1, buf_a, buf_b, sh0, sh1, ga, gb, w0, w1):
        wid = lax.axis_index("s") * NC + lax.axis_index("c")
        base = wid * (nchunk * ch)
        accs, shsems, wsems = (acc0, acc1), (sh0, sh1), (w0, w1)

        def add_into(acc, buf):
            def col(j, carry):
                for u in range(4):
                    s = pl.ds((j * 4 + u) * 16, 16)
                    for r in range(ch):
                        acc[r, s] = acc[r, s] + buf[r, s]
                return carry
            lax.fori_loop(0, ncol // 4, col, 0)

        pltpu.sync_copy(p0_hbm.at[wid], i0_v)
        pltpu.sync_copy(p1_hbm.at[wid], i1_v)
        pltpu.async_copy(sh_hbm.at[pl.ds(base, ch)], accs[0], shsems[0])
        pltpu.async_copy(ys_hbm.at[i0_v.at[0]], buf_a, ga)
        pltpu.async_copy(ys_hbm.at[i1_v.at[0]], buf_b, gb)
        for c in range(nchunk):
            acc = accs[c % 2]
            pltpu.make_async_copy(sh_hbm.at[pl.ds(base + c * ch, ch)], acc,
                                  shsems[c % 2]).wait()
            pltpu.make_async_copy(ys_hbm.at[i0_v.at[c]], buf_a, ga).wait()
            add_into(acc, buf_a)
            if c + 1 < nchunk:
                pltpu.async_copy(ys_hbm.at[i0_v.at[c + 1]], buf_a, ga)
            pltpu.make_async_copy(ys_hbm.at[i1_v.at[c]], buf_b, gb).wait()
            add_into(acc, buf_b)
            if c + 1 < nchunk:
                pltpu.async_copy(ys_hbm.at[i1_v.at[c + 1]], buf_b, gb)
                if c >= 1:
                    pltpu.make_async_copy(
                        accs[(c + 1) % 2],
                        out_hbm.at[pl.ds(base + (c - 1) * ch, ch)],
                        wsems[(c + 1) % 2]).wait()
                pltpu.async_copy(sh_hbm.at[pl.ds(base + (c + 1) * ch, ch)],
                                 accs[(c + 1) % 2], shsems[(c + 1) % 2])
            pltpu.async_copy(acc, out_hbm.at[pl.ds(base + c * ch, ch)],
                             wsems[c % 2])
        for c in (nchunk - 2, nchunk - 1):
            pltpu.make_async_copy(accs[c % 2],
                                  out_hbm.at[pl.ds(base + c * ch, ch)],
                                  wsems[c % 2]).wait()

    return k(shared, ys, pos0_3, pos1_3)


def _route(combine, selm):
    sel = selm > 0.5
    selj = sel.astype(jnp.int32)
    cnt_excl = jnp.cumsum(selj, axis=0) - selj
    counts = jnp.sum(selj, axis=0)
    tiles_e = (counts + TILE - 1) // TILE
    tile_end = jnp.cumsum(tiles_e)
    seg_start = (tile_end - tiles_e) * TILE
    dest = seg_start[None, :] + cnt_excl
    dest_full = jnp.where(sel, dest, GP)

    tokens = jnp.broadcast_to(jnp.arange(T, dtype=jnp.int32)[:, None], (T, E))
    flat_dest = dest_full.ravel()
    tok_ids = jnp.zeros((GP + 1,), jnp.int32).at[flat_dest].set(tokens.ravel())[:GP]
    wrow = jnp.zeros((GP + 1,), jnp.float32).at[flat_dest].set(combine.ravel())[:GP]

    pos2 = jnp.sort(dest_full, axis=1)[:, :K]
    pos0 = pos2[:, 0].astype(jnp.int32).reshape(NW, -1, 8)
    pos1 = pos2[:, 1].astype(jnp.int32).reshape(NW, -1, 8)

    tnums = jnp.arange(NT, dtype=jnp.int32)
    tid_raw = jnp.searchsorted(tile_end, tnums, side="right").astype(jnp.int32)
    used = tile_end[E - 1]
    last_tid = tid_raw[used - 1]
    tid = jnp.where(tnums < used, tid_raw, last_tid)
    return (tok_ids.reshape(NW, -1, 24), wrow.reshape(NT, 1, TILE),
            pos0, pos1, tid)


def kernel(x, gate_w, Wg, Wu, Wd, sg, su, sd):
    sg_b = sg.astype(jnp.bfloat16)
    su_b = su.astype(jnp.bfloat16)
    sd_b = sd.astype(jnp.bfloat16)
    wg_b = Wg.astype(jnp.bfloat16)
    wu_b = Wu.astype(jnp.bfloat16)
    wd_b = Wd.astype(jnp.bfloat16)

    shared, combine, selm = _k1(x, gate_w, sg_b, su_b, sd_b)
    tok_ids, wrow3, pos0, pos1, tid = _route(combine, selm)
    xs = _sc_gather(x, tok_ids)
    ys = _k3(tid, xs, wg_b, wu_b, wd_b, wrow3)
    return _sc_combine(shared, ys, pos0, pos1)


# speedup vs baseline: 1.0367x; 1.0034x over previous
import functools

import jax
import jax.numpy as jnp
from jax import lax
from jax.experimental import pallas as pl
from jax.experimental.pallas import tpu as pltpu
from jax.experimental.pallas import tpu_sc as plsc

T, H, I, E, K = 2048, 2048, 1536, 8, 2
TILE = 256
NT = (T * K) // TILE + E
GP = NT * TILE
NC, NS = 2, 16
NW = NC * NS


def _k1_body(x_ref, gw_ref, sg_ref, su_ref, sd_ref, sh_ref, cmb_ref, sel_ref):
    xf = x_ref[...]
    xb = xf.astype(jnp.bfloat16)
    g = lax.dot_general(xb, sg_ref[...], (((1,), (1,)), ((), ())),
                        preferred_element_type=jnp.float32)
    u = lax.dot_general(xb, su_ref[...], (((1,), (1,)), ((), ())),
                        preferred_element_type=jnp.float32)
    h = (g * jax.nn.sigmoid(g) * u).astype(jnp.bfloat16)
    sh_ref[...] = lax.dot_general(h, sd_ref[...], (((1,), (1,)), ((), ())),
                                  preferred_element_type=jnp.float32)

    logits = lax.dot_general(xb, gw_ref[...].astype(jnp.bfloat16),
                             (((1,), (1,)), ((), ())),
                             preferred_element_type=jnp.float32)
    idx = lax.broadcasted_iota(jnp.int32, (TILE, E), 1)
    m1 = jnp.max(logits, axis=1, keepdims=True)
    i1 = jnp.min(jnp.where(logits == m1, idx, E + 1), axis=1, keepdims=True)
    l2 = jnp.where(idx == i1, -jnp.inf, logits)
    m2 = jnp.max(l2, axis=1, keepdims=True)
    i2 = jnp.min(jnp.where(l2 == m2, idx, E + 1), axis=1, keepdims=True)
    w1 = 1.0 / (1.0 + jnp.exp(m2 - m1))
    w2 = 1.0 - w1
    on1 = (idx == i1)
    on2 = (idx == i2)
    cmb_ref[...] = jnp.where(on1, w1, 0.0) + jnp.where(on2, w2, 0.0)
    sel_ref[...] = jnp.where(on1 | on2, 1.0, 0.0)


def _k1(x, gate_w, sg_b, su_b, sd_b):
    return pl.pallas_call(
        _k1_body,
        grid=(T // TILE,),
        in_specs=[
            pl.BlockSpec((TILE, H), lambda i: (i, 0)),
            pl.BlockSpec((E, H), lambda i: (0, 0)),
            pl.BlockSpec((I, H), lambda i: (0, 0)),
            pl.BlockSpec((I, H), lambda i: (0, 0)),
            pl.BlockSpec((H, I), lambda i: (0, 0)),
        ],
        out_specs=[
            pl.BlockSpec((TILE, H), lambda i: (i, 0)),
            pl.BlockSpec((TILE, E), lambda i: (i, 0)),
            pl.BlockSpec((TILE, E), lambda i: (i, 0)),
        ],
        out_shape=[
            jax.ShapeDtypeStruct((T, H), jnp.float32),
            jax.ShapeDtypeStruct((T, E), jnp.float32),
            jax.ShapeDtypeStruct((T, E), jnp.float32),
        ],
    )(x, gate_w, sg_b, su_b, sd_b)


def _k3_body(tid_ref, used_ref, xs_ref, wg_ref, wu_ref, wd_ref, wr_ref, ys_ref):
    @pl.when(pl.program_id(0) < used_ref[0])
    def _():
        xb = xs_ref[...].astype(jnp.bfloat16)
        g = lax.dot_general(xb, wg_ref[0], (((1,), (1,)), ((), ())),
                            preferred_element_type=jnp.float32)
        u = lax.dot_general(xb, wu_ref[0], (((1,), (1,)), ((), ())),
                            preferred_element_type=jnp.float32)
        h = (g * jax.nn.sigmoid(g) * u).astype(jnp.bfloat16)
        y = lax.dot_general(h, wd_ref[0], (((1,), (1,)), ((), ())),
                            preferred_element_type=jnp.float32)
        ys_ref[...] = y * wr_ref[0, 0, :][:, None]


def _k3(tid, used, xsp, wg_b, wu_b, wd_b, wrow3):
    spec = pltpu.PrefetchScalarGridSpec(
        num_scalar_prefetch=2,
        grid=(NT,),
        in_specs=[
            pl.BlockSpec((TILE, H), lambda i, tid, used: (i, 0)),
            pl.BlockSpec((1, I, H), lambda i, tid, used: (tid[i], 0, 0)),
            pl.BlockSpec((1, I, H), lambda i, tid, used: (tid[i], 0, 0)),
            pl.BlockSpec((1, H, I), lambda i, tid, used: (tid[i], 0, 0)),
            pl.BlockSpec((1, 1, TILE), lambda i, tid, used: (i, 0, 0)),
        ],
        out_specs=pl.BlockSpec((TILE, H), lambda i, tid, used: (i, 0)),
    )
    return pl.pallas_call(
        _k3_body,
        grid_spec=spec,
        out_shape=jax.ShapeDtypeStruct((GP, H), jnp.float32),
    )(tid, used, xsp, wg_b, wu_b, wd_b, wrow3)


def _sc_gather(x, tok_ids3):
    ch = 16
    nchunk = (GP // NW) // ch
    nd = 3
    mesh = plsc.VectorSubcoreMesh(core_axis_name="c", subcore_axis_name="s")

    @functools.partial(
        pl.kernel, mesh=mesh,
        out_type=jax.ShapeDtypeStruct((GP, H), jnp.float32),
        scratch_types=[
            pltpu.VMEM((nchunk, ch), jnp.int32),
            pltpu.VMEM((ch, H), jnp.float32),
            pltpu.VMEM((ch, H), jnp.float32),
            pltpu.VMEM((ch, H), jnp.float32),
            pltpu.SemaphoreType.DMA,
            pltpu.SemaphoreType.DMA,
            pltpu.SemaphoreType.DMA,
            pltpu.SemaphoreType.DMA,
            pltpu.SemaphoreType.DMA,
            pltpu.SemaphoreType.DMA,
        ],
    )
    def k(x_hbm, ids_hbm, xs_hbm, idx_v, r0, r1, r2, g0, g1, g2, w0, w1, w2):
        wid = lax.axis_index("s") * NC + lax.axis_index("c")
        base = wid * (nchunk * ch)
        bufs, gsems, wsems = (r0, r1, r2), (g0, g1, g2), (w0, w1, w2)

        pltpu.sync_copy(ids_hbm.at[wid], idx_v)
        for c in range(nd):
            pltpu.async_copy(x_hbm.at[idx_v.at[c]], bufs[c % nd], gsems[c % nd])
        for c in range(nchunk):
            pltpu.make_async_copy(x_hbm.at[idx_v.at[c]], bufs[c % nd],
                                  gsems[c % nd]).wait()
            pltpu.async_copy(bufs[c % nd], xs_hbm.at[pl.ds(base + c * ch, ch)],
                             wsems[c % nd])
            if c + nd < nchunk:
                pltpu.make_async_copy(bufs[c % nd],
                                      xs_hbm.at[pl.ds(base + c * ch, ch)],
                                      wsems[c % nd]).wait()
                pltpu.async_copy(x_hbm.at[idx_v.at[c + nd]], bufs[c % nd],
                                 gsems[c % nd])
        for c in range(max(0, nchunk - nd), nchunk):
            pltpu.make_async_copy(bufs[c % nd],
                                  xs_hbm.at[pl.ds(base + c * ch, ch)],
                                  wsems[c % nd]).wait()

    return k(x, tok_ids3)


def _sc_combine(shared, ys, pos0_3, pos1_3):
    ch = 8
    nchunk = (T // NW) // ch
    ncol = H // 16
    mesh = plsc.VectorSubcoreMesh(core_axis_name="c", subcore_axis_name="s")

    @functools.partial(
        pl.kernel, mesh=mesh,
        out_type=jax.ShapeDtypeStruct((T, H), jnp.float32),
        scratch_types=[
            pltpu.VMEM((nchunk, ch), jnp.int32),
            pltpu.VMEM((nchunk, ch), jnp.int32),
            pltpu.VMEM((ch, H), jnp.float32),
            pltpu.VMEM((ch, H), jnp.float32),
            pltpu.VMEM((ch, H), jnp.float32),
            pltpu.VMEM((ch, H), jnp.float32),
            pltpu.SemaphoreType.DMA,
            pltpu.SemaphoreType.DMA,
            pltpu.SemaphoreType.DMA,
            pltpu.SemaphoreType.DMA,
            pltpu.SemaphoreType.DMA,
            pltpu.SemaphoreType.DMA,
        ],
    )
    def k(sh_hbm, ys_hbm, p0_hbm, p1_hbm, out_hbm, i0_v, i1_v,
          acc0, acc1, buf_a, buf_b, sh0, sh1, ga, gb, w0, w1):
        wid = lax.axis_index("s") * NC + lax.axis_index("c")
        base = wid * (nchunk * ch)
        accs, shsems, wsems = (acc0, acc1), (sh0, sh1), (w0, w1)

        def add_into(acc, buf):
            def col(j, carry):
                for u in range(4):
                    s = pl.ds((j * 4 + u) * 16, 16)
                    for r in range(ch):
                        acc[r, s] = acc[r, s] + buf[r, s]
                return carry
            lax.fori_loop(0, ncol // 4, col, 0)

        pltpu.sync_copy(p0_hbm.at[wid], i0_v)
        pltpu.sync_copy(p1_hbm.at[wid], i1_v)
        pltpu.async_copy(sh_hbm.at[pl.ds(base, ch)], accs[0], shsems[0])
        pltpu.async_copy(ys_hbm.at[i0_v.at[0]], buf_a, ga)
        pltpu.async_copy(ys_hbm.at[i1_v.at[0]], buf_b, gb)
        for c in range(nchunk):
            acc = accs[c % 2]
            pltpu.make_async_copy(sh_hbm.at[pl.ds(base + c * ch, ch)], acc,
                                  shsems[c % 2]).wait()
            pltpu.make_async_copy(ys_hbm.at[i0_v.at[c]], buf_a, ga).wait()
            add_into(acc, buf_a)
            if c + 1 < nchunk:
                pltpu.async_copy(ys_hbm.at[i0_v.at[c + 1]], buf_a, ga)
            pltpu.make_async_copy(ys_hbm.at[i1_v.at[c]], buf_b, gb).wait()
            add_into(acc, buf_b)
            if c + 1 < nchunk:
                pltpu.async_copy(ys_hbm.at[i1_v.at[c + 1]], buf_b, gb)
                if c >= 1:
                    pltpu.make_async_copy(
                        accs[(c + 1) % 2],
                        out_hbm.at[pl.ds(base + (c - 1) * ch, ch)],
                        wsems[(c + 1) % 2]).wait()
                pltpu.async_copy(sh_hbm.at[pl.ds(base + (c + 1) * ch, ch)],
                                 accs[(c + 1) % 2], shsems[(c + 1) % 2])
            pltpu.async_copy(acc, out_hbm.at[pl.ds(base + c * ch, ch)],
                             wsems[c % 2])
        for c in (nchunk - 2, nchunk - 1):
            pltpu.make_async_copy(accs[c % 2],
                                  out_hbm.at[pl.ds(base + c * ch, ch)],
                                  wsems[c % 2]).wait()

    return k(shared, ys, pos0_3, pos1_3)


def _route(combine, selm):
    sel = selm > 0.5
    selj = sel.astype(jnp.int32)
    cnt_excl = jnp.cumsum(selj, axis=0) - selj
    counts = jnp.sum(selj, axis=0)
    tiles_e = (counts + TILE - 1) // TILE
    tile_end = jnp.cumsum(tiles_e)
    seg_start = (tile_end - tiles_e) * TILE
    dest = seg_start[None, :] + cnt_excl
    dest_full = jnp.where(sel, dest, GP)

    tokens = jnp.broadcast_to(jnp.arange(T, dtype=jnp.int32)[:, None], (T, E))
    flat_dest = dest_full.ravel()
    tok_ids = jnp.zeros((GP + 1,), jnp.int32).at[flat_dest].set(tokens.ravel())[:GP]
    wrow = jnp.zeros((GP + 1,), jnp.float32).at[flat_dest].set(combine.ravel())[:GP]

    pos2 = jnp.sort(dest_full, axis=1)[:, :K]
    pos0 = pos2[:, 0].astype(jnp.int32).reshape(NW, -1, 8)
    pos1 = pos2[:, 1].astype(jnp.int32).reshape(NW, -1, 8)

    tnums = jnp.arange(NT, dtype=jnp.int32)
    tid_raw = jnp.searchsorted(tile_end, tnums, side="right").astype(jnp.int32)
    used = tile_end[E - 1]
    last_tid = tid_raw[used - 1]
    tid = jnp.where(tnums < used, tid_raw, last_tid)
    return (tok_ids.reshape(NW, -1, 16), wrow.reshape(NT, 1, TILE),
            pos0, pos1, tid, tile_end[E - 1:E].astype(jnp.int32))


def kernel(x, gate_w, Wg, Wu, Wd, sg, su, sd):
    sg_b = sg.astype(jnp.bfloat16)
    su_b = su.astype(jnp.bfloat16)
    sd_b = sd.astype(jnp.bfloat16)
    wg_b = Wg.astype(jnp.bfloat16)
    wu_b = Wu.astype(jnp.bfloat16)
    wd_b = Wd.astype(jnp.bfloat16)

    shared, combine, selm = _k1(x, gate_w, sg_b, su_b, sd_b)
    tok_ids, wrow3, pos0, pos1, tid, used = _route(combine, selm)
    xs = _sc_gather(x, tok_ids)
    ys = _k3(tid, used, xs, wg_b, wu_b, wd_b, wrow3)
    return _sc_combine(shared, ys, pos0, pos1)
